# baseline (speedup 1.0000x reference)
import functools

import jax
import jax.numpy as jnp
from jax import lax
from jax.experimental import pallas as pl
from jax.experimental.pallas import tpu as pltpu
from jax.experimental.pallas import tpu_sc as plsc

N = 50000
E = 1600000
IN_DIM = 128
OUT_DIM = 32
NUM_LAYERS = 3

NC = 2
NS = 16
NW = NC * NS
EPW = E // NW
SUB = 125
RPC = 2
CHUNK = SUB * RPC
NCHUNKS = EPW // CHUNK
NPAIR = NCHUNKS // 2
EROWS = E // SUB
RPW = EPW // SUB
NPS = 3128
N_PAD = NS * NPS
DEGW = 8

_MESH = plsc.VectorSubcoreMesh(core_axis_name="c", subcore_axis_name="s")
_SC_PARAMS = pltpu.CompilerParams(use_tc_tiling_on_sc=False)

_ROW_BYTES = SUB * OUT_DIM * 4
_DEG_BYTES = SUB * DEGW * 4


@functools.partial(
    pl.kernel,
    out_type=jax.ShapeDtypeStruct((NC, N_PAD, OUT_DIM), jnp.float32),
    mesh=_MESH,
    compiler_params=_SC_PARAMS,
    scratch_types=[
        pltpu.VMEM((2, 1, CHUNK), jnp.int32),
        pltpu.VMEM((2, 1, CHUNK), jnp.int32),
        pltpu.VMEM((2, CHUNK, OUT_DIM), jnp.float32),
        pltpu.VMEM_SHARED((N_PAD, OUT_DIM), jnp.float32),
        pltpu.SemaphoreType.DMA,
        pltpu.SemaphoreType.DMA,
        pltpu.SemaphoreType.DMA,
        pltpu.SemaphoreType.DMA,
    ],
)
def _sc_aggregate(h_hbm, srcf_hbm, dstf_hbm, zeros_hbm, out_hbm,
                  src_v, dst_v, rows_v, acc_sh, gsem0, gsem1, ssem0, ssem1):
    c = lax.axis_index("c")
    s = lax.axis_index("s")
    wid = c * NS + s
    gsems = (gsem0, gsem1)
    ssems = (ssem0, ssem1)

    pltpu.sync_copy(zeros_hbm, acc_sh.at[pl.ds(s * NPS, NPS)])
    plsc.subcore_barrier()

    def pair_body(i, carry):
        gathers = []
        for b in range(2):
            base_row = wid * RPW + (2 * i + b) * RPC
            @pl.when(i > 0)
            def _(b=b):
                pltpu.make_async_copy(
                    h_hbm.at[pl.ds(0, CHUNK)], rows_v.at[b], ssems[b]).wait()
            pltpu.sync_copy(
                srcf_hbm.at[pl.ds(wid * NCHUNKS + 2 * i + b, 1)], src_v.at[b])
            pltpu.sync_copy(
                dstf_hbm.at[pl.ds(wid * NCHUNKS + 2 * i + b, 1)], dst_v.at[b])
            gathers.append(pltpu.async_copy(
                h_hbm.at[src_v.at[b, 0]], rows_v.at[b], gsems[b]))
        for b in range(2):
            gathers[b].wait()
            pltpu.async_copy(rows_v.at[b], acc_sh.at[dst_v.at[b, 0]],
                             ssems[b], add=True)
        return carry

    lax.fori_loop(0, NPAIR, pair_body, 0)
    for b in range(2):
        pltpu.make_async_copy(h_hbm.at[pl.ds(0, CHUNK)], rows_v.at[b],
                              ssems[b]).wait()
    plsc.subcore_barrier()

    pltpu.sync_copy(acc_sh.at[pl.ds(s * NPS, NPS)],
                    out_hbm.at[c, pl.ds(s * NPS, NPS)])


@functools.partial(
    pl.kernel,
    out_type=jax.ShapeDtypeStruct((NC, N_PAD, DEGW), jnp.float32),
    mesh=_MESH,
    compiler_params=_SC_PARAMS,
    scratch_types=[
        pltpu.VMEM((2, RPC, SUB), jnp.int32),
        pltpu.VMEM((SUB, DEGW), jnp.float32),
        pltpu.VMEM_SHARED((N_PAD, DEGW), jnp.float32),
        pltpu.SemaphoreType.DMA,
        pltpu.SemaphoreType.DMA,
    ],
)
def _sc_degree(dst_hbm, ones_hbm, zeros_hbm, out_hbm,
               dst_v, ones_v, acc_sh, ssem0, ssem1):
    c = lax.axis_index("c")
    s = lax.axis_index("s")
    wid = c * NS + s
    ssems = (ssem0, ssem1)

    pltpu.sync_copy(ones_hbm, ones_v)
    pltpu.sync_copy(zeros_hbm, acc_sh.at[pl.ds(s * NPS, NPS)])
    plsc.subcore_barrier()

    def pair_body(i, carry):
        for b in range(2):
            base_row = wid * RPW + (2 * i + b) * RPC
            @pl.when(i > 0)
            def _(b=b):
                for j in range(RPC):
                    pltpu.make_async_copy(ones_hbm, ones_v, ssems[b]).wait()
            pltpu.sync_copy(dst_hbm.at[pl.ds(base_row, RPC)], dst_v.at[b])
            for j in range(RPC):
                pltpu.async_copy(ones_v, acc_sh.at[dst_v.at[b, j]],
                                 ssems[b], add=True)
        return carry

    lax.fori_loop(0, NPAIR, pair_body, 0)
    for b in range(2):
        for j in range(RPC):
            pltpu.make_async_copy(ones_hbm, ones_v, ssems[b]).wait()
    plsc.subcore_barrier()

    pltpu.sync_copy(acc_sh.at[pl.ds(s * NPS, NPS)],
                    out_hbm.at[c, pl.ds(s * NPS, NPS)])


_BLK = 2000


def _lin_in_body(x_ref, w_ref, b_ref, out_ref):
    out_ref[...] = jnp.dot(x_ref[...], w_ref[...],
                           preferred_element_type=jnp.float32) + b_ref[...]


def _lin_in(x, w_t, b):
    return pl.pallas_call(
        _lin_in_body,
        grid=(N // _BLK,),
        in_specs=[
            pl.BlockSpec((_BLK, IN_DIM), lambda i: (i, 0)),
            pl.BlockSpec((IN_DIM, OUT_DIM), lambda i: (0, 0)),
            pl.BlockSpec((1, OUT_DIM), lambda i: (0, 0)),
        ],
        out_specs=pl.BlockSpec((_BLK, OUT_DIM), lambda i: (i, 0)),
        out_shape=jax.ShapeDtypeStruct((N, OUT_DIM), jnp.float32),
    )(x, w_t, b)


def _combine_body(a0_ref, a1_ref, d0_ref, d1_ref, h_ref,
                  wl_ref, wr_ref, bl_ref, g_ref, b_ref, out_ref):
    deg = jnp.maximum(d0_ref[0, :, 0:1] + d1_ref[0, :, 0:1], 1.0)
    aggr = (a0_ref[0] + a1_ref[0]) / deg
    h2 = (jnp.dot(aggr, wl_ref[...], preferred_element_type=jnp.float32)
          + jnp.dot(h_ref[...], wr_ref[...], preferred_element_type=jnp.float32)
          + bl_ref[...])
    h2 = jnp.maximum(h2, 0.0)
    mu = jnp.mean(h2, axis=-1, keepdims=True)
    var = jnp.mean((h2 - mu) ** 2, axis=-1, keepdims=True)
    out_ref[...] = ((h2 - mu) * lax.rsqrt(var + 1e-5) * g_ref[...]
                    + b_ref[...])


def _combine(parts, degs, h, wl_t, wr_t, bl_i, gamma, beta):
    row_spec = pl.BlockSpec((_BLK, OUT_DIM), lambda i: (i, 0))
    par_spec = pl.BlockSpec((1, OUT_DIM), lambda i: (0, 0))
    p0_spec = pl.BlockSpec((1, _BLK, OUT_DIM), lambda i: (0, i, 0))
    p1_spec = pl.BlockSpec((1, _BLK, OUT_DIM), lambda i: (1, i, 0))
    d0_spec = pl.BlockSpec((1, _BLK, DEGW), lambda i: (0, i, 0))
    d1_spec = pl.BlockSpec((1, _BLK, DEGW), lambda i: (1, i, 0))
    return pl.pallas_call(
        _combine_body,
        grid=(N // _BLK,),
        in_specs=[
            p0_spec, p1_spec, d0_spec, d1_spec, row_spec,
            pl.BlockSpec((OUT_DIM, OUT_DIM), lambda i: (0, 0)),
            pl.BlockSpec((OUT_DIM, OUT_DIM), lambda i: (0, 0)),
            par_spec, par_spec, par_spec,
        ],
        out_specs=row_spec,
        out_shape=jax.ShapeDtypeStruct((N, OUT_DIM), jnp.float32),
    )(parts, parts, degs, degs, h, wl_t, wr_t, bl_i, gamma, beta)


def kernel(x, edge_index, W_in, b_in, Wl, bl, Wr, gamma, beta):
    src_flat = edge_index[0].reshape(NW * NCHUNKS, CHUNK)
    dst_flat = edge_index[1].reshape(NW * NCHUNKS, CHUNK)
    dst_rows = edge_index[1].reshape(EROWS, SUB)

    zeros_slab = jnp.zeros((NPS, OUT_DIM), dtype=jnp.float32)
    zeros_deg = jnp.zeros((NPS, DEGW), dtype=jnp.float32)
    ones_col = jnp.ones((SUB, DEGW), dtype=jnp.float32)

    h = _lin_in(x, W_in.T, b_in.reshape(1, OUT_DIM))

    degs = _sc_degree(dst_rows, ones_col, zeros_deg)

    for i in range(NUM_LAYERS):
        parts = _sc_aggregate(h, src_flat, dst_flat, zeros_slab)
        h = _combine(parts, degs, h,
                     Wl[i].T, Wr[i].T, bl[i].reshape(1, OUT_DIM),
                     gamma.reshape(1, OUT_DIM), beta.reshape(1, OUT_DIM))
    return h


# speedup vs baseline: 1.1755x; 1.1755x over previous
import functools

import jax
import jax.numpy as jnp
from jax import lax
from jax.experimental import pallas as pl
from jax.experimental.pallas import tpu as pltpu
from jax.experimental.pallas import tpu_sc as plsc

N = 50000
E = 1600000
IN_DIM = 128
OUT_DIM = 32
NUM_LAYERS = 3

NC = 2
NS = 16
NW = NC * NS
EPW = E // NW
SUB = 125
RPC = 2
CHUNK = SUB * RPC
NCHUNKS = EPW // CHUNK
NPAIR = NCHUNKS // 2
EROWS = E // SUB
RPW = EPW // SUB
NPS = 3200
N_PAD = NS * NPS
DEGW = 32

_MESH = plsc.VectorSubcoreMesh(core_axis_name="c", subcore_axis_name="s")
_SC_PARAMS = pltpu.CompilerParams(use_tc_tiling_on_sc=False)

_ROW_BYTES = SUB * OUT_DIM * 4
_DEG_BYTES = SUB * DEGW * 4


@functools.partial(
    pl.kernel,
    out_type=jax.ShapeDtypeStruct((NC, N_PAD, OUT_DIM), jnp.float32),
    mesh=_MESH,
    compiler_params=_SC_PARAMS,
    scratch_types=[
        pltpu.VMEM((2, 1, CHUNK), jnp.int32),
        pltpu.VMEM((2, RPC, SUB), jnp.int32),
        pltpu.VMEM((2, CHUNK, OUT_DIM), jnp.float32),
        pltpu.VMEM_SHARED((N_PAD, OUT_DIM), jnp.float32),
        pltpu.SemaphoreType.DMA,
        pltpu.SemaphoreType.DMA,
        pltpu.SemaphoreType.DMA,
        pltpu.SemaphoreType.DMA,
    ],
)
def _sc_aggregate(h_hbm, srcf_hbm, dst_hbm, zeros_hbm, out_hbm,
                  src_v, dst_v, rows_v, acc_sh, gsem0, gsem1, ssem0, ssem1):
    c = lax.axis_index("c")
    s = lax.axis_index("s")
    wid = c * NS + s
    gsems = (gsem0, gsem1)
    ssems = (ssem0, ssem1)

    pltpu.sync_copy(zeros_hbm, acc_sh.at[pl.ds(s * NPS, NPS)])
    plsc.subcore_barrier()

    def pair_body(i, carry):
        gathers = []
        for b in range(2):
            base_row = wid * RPW + (2 * i + b) * RPC
            @pl.when(i > 0)
            def _(b=b):
                for j in range(RPC):
                    pltpu.make_async_copy(
                        zeros_hbm.at[pl.ds(0, SUB)],
                        rows_v.at[b, pl.ds(j * SUB, SUB)], ssems[b]).wait()
            pltpu.sync_copy(
                srcf_hbm.at[pl.ds(wid * NCHUNKS + 2 * i + b, 1)], src_v.at[b])
            pltpu.sync_copy(dst_hbm.at[pl.ds(base_row, RPC)], dst_v.at[b])
            gathers.append(pltpu.async_copy(
                h_hbm.at[src_v.at[b, 0]], rows_v.at[b], gsems[b]))
        for b in range(2):
            gathers[b].wait()
            for j in range(RPC):
                pltpu.async_copy(rows_v.at[b, pl.ds(j * SUB, SUB)],
                                 acc_sh.at[dst_v.at[b, j]], ssems[b],
                                 add=True)
        return carry

    lax.fori_loop(0, NPAIR, pair_body, 0)
    for b in range(2):
        for j in range(RPC):
            pltpu.make_async_copy(zeros_hbm.at[pl.ds(0, SUB)],
                                  rows_v.at[b, pl.ds(j * SUB, SUB)],
                                  ssems[b]).wait()
    plsc.subcore_barrier()

    pltpu.sync_copy(acc_sh.at[pl.ds(s * NPS, NPS)],
                    out_hbm.at[c, pl.ds(s * NPS, NPS)])


@functools.partial(
    pl.kernel,
    out_type=jax.ShapeDtypeStruct((NC, N_PAD, DEGW), jnp.float32),
    mesh=_MESH,
    compiler_params=_SC_PARAMS,
    scratch_types=[
        pltpu.VMEM((2, RPC, SUB), jnp.int32),
        pltpu.VMEM((SUB, DEGW), jnp.float32),
        pltpu.VMEM_SHARED((N_PAD, DEGW), jnp.float32),
        pltpu.SemaphoreType.DMA,
        pltpu.SemaphoreType.DMA,
    ],
)
def _sc_degree(dst_hbm, ones_hbm, zeros_hbm, out_hbm,
               dst_v, ones_v, acc_sh, ssem0, ssem1):
    c = lax.axis_index("c")
    s = lax.axis_index("s")
    wid = c * NS + s
    ssems = (ssem0, ssem1)

    pltpu.sync_copy(ones_hbm, ones_v)
    pltpu.sync_copy(zeros_hbm, acc_sh.at[pl.ds(s * NPS, NPS)])
    plsc.subcore_barrier()

    def pair_body(i, carry):
        for b in range(2):
            base_row = wid * RPW + (2 * i + b) * RPC
            @pl.when(i > 0)
            def _(b=b):
                for j in range(RPC):
                    pltpu.make_async_copy(ones_hbm, ones_v, ssems[b]).wait()
            pltpu.sync_copy(dst_hbm.at[pl.ds(base_row, RPC)], dst_v.at[b])
            for j in range(RPC):
                pltpu.async_copy(ones_v, acc_sh.at[dst_v.at[b, j]],
                                 ssems[b], add=True)
        return carry

    lax.fori_loop(0, NPAIR, pair_body, 0)
    for b in range(2):
        for j in range(RPC):
            pltpu.make_async_copy(ones_hbm, ones_v, ssems[b]).wait()
    plsc.subcore_barrier()

    pltpu.sync_copy(acc_sh.at[pl.ds(s * NPS, NPS)],
                    out_hbm.at[c, pl.ds(s * NPS, NPS)])


_BLK = 400
_PBLK = _BLK // 4
_PN = N * OUT_DIM // 128
_PCORE = N_PAD * OUT_DIM // 128


def _lin_in_body(x_ref, w_ref, b_ref, out_ref):
    out_ref[...] = jnp.dot(x_ref[...], w_ref[...],
                           preferred_element_type=jnp.float32) + b_ref[...]


def _lin_in(x, w_t, b):
    return pl.pallas_call(
        _lin_in_body,
        grid=(N // _BLK,),
        in_specs=[
            pl.BlockSpec((_BLK, IN_DIM), lambda i: (i, 0)),
            pl.BlockSpec((IN_DIM, OUT_DIM), lambda i: (0, 0)),
            pl.BlockSpec((1, OUT_DIM), lambda i: (0, 0)),
        ],
        out_specs=pl.BlockSpec((_BLK, OUT_DIM), lambda i: (i, 0)),
        out_shape=jax.ShapeDtypeStruct((N, OUT_DIM), jnp.float32),
    )(x, w_t, b)


def _combine_body(a0_ref, a1_ref, d0_ref, d1_ref, h_ref,
                  wl_ref, wr_ref, m_ref, bl_ref, g_ref, b_ref, out_ref):
    deg = jnp.maximum(d0_ref[0:_PN] + d1_ref[0:_PN], 1.0)
    aggr = (a0_ref[0:_PN] + a1_ref[0:_PN]) / deg
    h2 = (jnp.dot(aggr, wl_ref[...], preferred_element_type=jnp.float32)
          + jnp.dot(h_ref[...], wr_ref[...], preferred_element_type=jnp.float32)
          + bl_ref[...])
    h2 = jnp.maximum(h2, 0.0)
    mu = jnp.dot(h2, m_ref[...], preferred_element_type=jnp.float32)
    cen = h2 - mu
    var = jnp.dot(cen * cen, m_ref[...], preferred_element_type=jnp.float32)
    out_ref[...] = cen * lax.rsqrt(var + 1e-5) * g_ref[...] + b_ref[...]


def _combine(parts_p, degs_p, h_p, wl4, wr4, m4, bl4, g4, b4):
    p0_spec = pl.BlockSpec((_PCORE, 128), lambda i: (0, 0))
    p1_spec = pl.BlockSpec((_PCORE, 128), lambda i: (1, 0))
    h_spec = pl.BlockSpec((_PN, 128), lambda i: (0, 0))
    w_spec = pl.BlockSpec((128, 128), lambda i: (0, 0))
    par_spec = pl.BlockSpec((1, 128), lambda i: (0, 0))
    return pl.pallas_call(
        _combine_body,
        grid=(1,),
        in_specs=[
            p0_spec, p1_spec, p0_spec, p1_spec, h_spec,
            w_spec, w_spec, w_spec,
            par_spec, par_spec, par_spec,
        ],
        out_specs=h_spec,
        out_shape=jax.ShapeDtypeStruct((_PN, 128), jnp.float32),
    )(parts_p, parts_p, degs_p, degs_p, h_p, wl4, wr4, m4, bl4, g4, b4)


def kernel(x, edge_index, W_in, b_in, Wl, bl, Wr, gamma, beta):
    src_flat = edge_index[0].reshape(NW * NCHUNKS, CHUNK)
    dst_rows = edge_index[1].reshape(EROWS, SUB)

    zeros_slab = jnp.zeros((NPS, OUT_DIM), dtype=jnp.float32)
    ones_col = jnp.ones((SUB, DEGW), dtype=jnp.float32)
    eye4 = jnp.eye(4, dtype=jnp.float32)
    m4 = jnp.kron(eye4, jnp.full((OUT_DIM, OUT_DIM), 1.0 / OUT_DIM,
                                 dtype=jnp.float32))
    g4 = jnp.tile(gamma, 4).reshape(1, 128)
    b4 = jnp.tile(beta, 4).reshape(1, 128)

    h = _lin_in(x, W_in.T, b_in.reshape(1, OUT_DIM))

    degs = _sc_degree(dst_rows, ones_col, zeros_slab)
    degs_p = degs.reshape(2 * _PCORE, 128)

    for i in range(NUM_LAYERS):
        parts = _sc_aggregate(h, src_flat, dst_rows, zeros_slab)
        h_p = _combine(parts.reshape(2 * _PCORE, 128), degs_p,
                       h.reshape(_PN, 128),
                       jnp.kron(eye4, Wl[i].T), jnp.kron(eye4, Wr[i].T), m4,
                       jnp.tile(bl[i], 4).reshape(1, 128), g4, b4)
        h = h_p.reshape(N, OUT_DIM)
    return h
